# baseline (speedup 1.0000x reference)
import functools

import jax
import jax.numpy as jnp
from jax import lax
from jax.experimental import pallas as pl
from jax.experimental.pallas import tpu as pltpu
from jax.experimental.pallas import tpu_sc as plsc

B = 128
N = 100000
NPAD = 100096
K = 5
L = 16
NS = 16
CH = 5888
NCHUNK = NPAD // CH
G = 16
GROUPS = CH // (L * G)
TAIL_VEC = (N - (NCHUNK - 1) * CH) // L
NEG = -3.0e38

_mesh = plsc.VectorSubcoreMesh(
    core_axis_name="c", subcore_axis_name="s", num_cores=1)


@functools.partial(
    pl.kernel,
    mesh=_mesh,
    out_type=jax.ShapeDtypeStruct((NS * L,), jnp.float32),
    scratch_types=[
        pltpu.VMEM((8, CH), jnp.float32),
        pltpu.VMEM((8, CH), jnp.float32),
        pltpu.VMEM((8, 128), jnp.float32),
        pltpu.VMEM((2 * L,), jnp.int32),
        pltpu.VMEM((2 * L,), jnp.float32),
        pltpu.VMEM((L,), jnp.float32),
        pltpu.VMEM((56, L), jnp.float32),
        pltpu.SemaphoreType.DMA,
        pltpu.SemaphoreType.DMA,
    ],
)
def _loss_sc(x_hbm, y_hbm, out_hbm, buf0, buf1, tbuf, ybuf, xbuf, obuf,
             accb, sem0, sem1):
    sid = lax.axis_index("s")
    rg = sid
    lane_ids = lax.iota(jnp.int32, L)
    row_base = pl.multiple_of(rg * 8, 8)

    gdims = lax.GatherDimensionNumbers(
        offset_dims=(), collapsed_slice_dims=(0,), start_index_map=(0,))

    def shuf(v, s):
        return lax.gather(
            v, (lane_ids ^ s)[:, None], gdims, (1,),
            mode=lax.GatherScatterMode.PROMISE_IN_BOUNDS)

    def bmax(v):
        for s in (1, 2, 4, 8):
            v = jnp.maximum(v, shuf(v, s))
        return v

    def bmin(v):
        for s in (1, 2, 4, 8):
            v = jnp.minimum(v, shuf(v, s))
        return v

    def insert(a, t):
        for k in range(6):
            hi = jnp.maximum(a[k], t)
            t = jnp.minimum(a[k], t)
            a[k] = hi

    def pop6(a):
        ms = []
        for j in range(6):
            m_v = bmax(a[0])
            ms.append(m_v)
            if j < 5:
                eq = a[0] == m_v
                pm = lane_ids == bmin(jnp.where(eq, lane_ids, L))
                for k in range(5):
                    a[k] = jnp.where(pm, a[k + 1], a[k])
                a[5] = jnp.where(pm, jnp.full((L,), NEG, jnp.float32), a[5])
        return ms

    def dma_chunk(c, buf, sem):
        return pltpu.async_copy(
            x_hbm.at[pl.ds(row_base, 8),
                     pl.ds(pl.multiple_of(c * CH, 128), CH)],
            buf, sem)

    def wait_chunk(c, buf, sem):
        pltpu.make_async_copy(
            x_hbm.at[pl.ds(row_base, 8),
                     pl.ds(pl.multiple_of(c * CH, 128), CH)],
            buf, sem).wait()

    def group_body(buf, rib, sb, i, dummy):
        base = i * (L * G)
        vs = [buf[rib, pl.ds(base + u * L, L)] for u in range(G)]
        gm = vs[0]
        for u in range(1, G):
            gm = jnp.maximum(gm, vs[u])
        trig = bmax(gm)[0] > accb[sb + 6][0]

        @pl.when(trig)
        def _():
            a = [accb[sb + k] for k in range(6)]
            for v in vs:
                insert(a, v)
            for k in range(6):
                accb[sb + k] = a[k]
            accb[sb + 6] = pop6(a)[5]

        return dummy

    ybase = pl.multiple_of((rg >> 1) * L, 16)
    pltpu.sync_copy(y_hbm.at[pl.ds(ybase, L)], ybuf.at[pl.ds(0, L)])

    neg_v = jnp.full((L,), NEG, jnp.float32)
    for sb in range(56):
        accb[sb] = neg_v

    def process(buf, rib, sb, ngroups):
        lax.fori_loop(
            0, ngroups, functools.partial(group_body, buf, rib, sb), 0)

    def rows(buf, ngroups):
        def row_body(r, dummy):
            process(buf, r, r * 7, ngroups)
            return dummy
        lax.fori_loop(0, 8, row_body, 0)

    dma_chunk(0, buf0, sem0)

    def ring_body(i, dummy):
        wait_chunk(2 * i, buf0, sem0)
        dma_chunk(2 * i + 1, buf1, sem1)
        rows(buf0, GROUPS)
        wait_chunk(2 * i + 1, buf1, sem1)
        dma_chunk(2 * i + 2, buf0, sem0)
        rows(buf1, GROUPS)
        return dummy

    lax.fori_loop(0, (NCHUNK - 1) // 2, ring_body, 0)

    wait_chunk(NCHUNK - 1, buf0, sem0)
    rows(buf0, TAIL_VEC // G)

    def tail_body(r, dummy):
        a = [accb[r * 7 + k] for k in range(6)]
        for v in range((TAIL_VEC // G) * G, TAIL_VEC):
            insert(a, buf0[r, pl.ds(v * L, L)])
        for k in range(6):
            accb[r * 7 + k] = a[k]
        return dummy
    lax.fori_loop(0, 8, tail_body, 0)

    acc_v = jnp.zeros((L,), jnp.float32)
    for r in range(8):
        lane = ((rg & 1) * 8 + r).astype(jnp.int32)
        y_val = ybuf[pl.ds(lane, L)][0]
        pltpu.sync_copy(
            x_hbm.at[pl.ds(row_base, 8),
                     pl.ds(pl.multiple_of((y_val >> 7) << 7, 128), 128)],
            tbuf)
        ylo = y_val & 127
        st16 = (ylo >> 4) << 4
        xbuf[pl.ds(0, L)] = tbuf[r, pl.ds(st16, L)]
        s_v = jnp.broadcast_to(xbuf[pl.ds(ylo & 15, L)][0], (L,))

        ms = pop6([accb[r * 7 + k] for k in range(6)])
        csum_v = jnp.zeros((L,), jnp.float32)
        for m_v in ms:
            csum_v = csum_v + jnp.maximum(m_v + 1.0 - s_v, 0.0)
        c6_v = jnp.maximum(ms[5] + 1.0 - s_v, 0.0)
        sub_v = jnp.where(s_v >= ms[5], jnp.full((L,), jnp.float32(1.0)),
                          c6_v)
        acc_v = acc_v + (csum_v - sub_v)

    obuf[...] = acc_v * jnp.float32(1.0 / (K * B))
    pltpu.sync_copy(obuf, out_hbm.at[pl.ds(pl.multiple_of(sid * L, 8), L)])


def kernel(x, y):
    parts = _loss_sc(x, y.astype(jnp.int32))
    return jnp.sum(parts.reshape(NS, L)[:, 0])


# speedup vs baseline: 1.6844x; 1.6844x over previous
import functools

import jax
import jax.numpy as jnp
from jax import lax
from jax.experimental import pallas as pl
from jax.experimental.pallas import tpu as pltpu
from jax.experimental.pallas import tpu_sc as plsc

B = 128
N = 100000
NPAD = 100096
K = 5
L = 16
NC = 2
NS = 16
NW = NC * NS
CH = 5888
NCHUNK = NPAD // CH
U = 8
FULL_IT = CH // (L * U)
TAIL_VEC = (N - (NCHUNK - 1) * CH) // L
NEG = -3.0e38

_mesh = plsc.VectorSubcoreMesh(core_axis_name="c", subcore_axis_name="s")


@functools.partial(
    pl.kernel,
    mesh=_mesh,
    out_type=jax.ShapeDtypeStruct((NW * L,), jnp.float32),
    scratch_types=[
        pltpu.VMEM((8, CH), jnp.float32),
        pltpu.VMEM((8, CH), jnp.float32),
        pltpu.VMEM((8, 128), jnp.float32),
        pltpu.VMEM((2 * L,), jnp.int32),
        pltpu.VMEM((2 * L,), jnp.float32),
        pltpu.VMEM((L,), jnp.float32),
        pltpu.SemaphoreType.DMA,
        pltpu.SemaphoreType.DMA,
    ],
)
def _loss_sc(x_hbm, y_hbm, out_hbm, buf0, buf1, tbuf, ybuf, xbuf, obuf,
             sem0, sem1):
    bufs = (buf0, buf1)
    sems = (sem0, sem1)
    cid = lax.axis_index("c")
    sid = lax.axis_index("s")
    wid = sid * NC + cid
    rg = cid * 8 + (sid >> 1)
    h = sid & 1
    lane_ids = lax.iota(jnp.int32, L)

    gdims = lax.GatherDimensionNumbers(
        offset_dims=(), collapsed_slice_dims=(0,), start_index_map=(0,))

    def shuf(v, s):
        return lax.gather(
            v, (lane_ids ^ s)[:, None], gdims, (1,),
            mode=lax.GatherScatterMode.PROMISE_IN_BOUNDS)

    def bmax(v):
        for s in (1, 2, 4, 8):
            v = jnp.maximum(v, shuf(v, s))
        return v

    def bmin(v):
        for s in (1, 2, 4, 8):
            v = jnp.minimum(v, shuf(v, s))
        return v

    ybase = pl.multiple_of((rg >> 1) * L, 16)
    pltpu.sync_copy(y_hbm.at[pl.ds(ybase, L)], ybuf.at[pl.ds(0, L)])

    def insert(a, t):
        for k in range(6):
            hi = jnp.maximum(a[k], t)
            t = jnp.minimum(a[k], t)
            a[k] = hi

    def chunk_body(pb, rib, i, carry):
        a = list(carry)
        base = i * (L * U)
        for u in range(U):
            insert(a, bufs[pb][rib, pl.ds(base + u * L, L)])
        return tuple(a)

    row_base = pl.multiple_of(rg * 8, 8)
    accs = [[jnp.full((L,), NEG, jnp.float32)] * 6 for _ in range(4)]

    pend = [None, None]
    pend[0] = pltpu.async_copy(
        x_hbm.at[pl.ds(row_base, 8), pl.ds(0, CH)], buf0, sems[0])
    for ci in range(NCHUNK):
        pb = ci % 2
        if ci + 1 < NCHUNK:
            nb = (ci + 1) % 2
            pend[nb] = pltpu.async_copy(
                x_hbm.at[pl.ds(row_base, 8),
                         pl.ds(pl.multiple_of((ci + 1) * CH, 128), CH)],
                bufs[nb], sems[nb])
        pend[pb].wait()
        n_it = FULL_IT if ci + 1 < NCHUNK else TAIL_VEC // U
        for r in range(4):
            rib = h * 4 + r
            accs[r] = list(lax.fori_loop(
                0, n_it, functools.partial(chunk_body, pb, rib),
                tuple(accs[r])))
            if ci + 1 == NCHUNK:
                for v in range(n_it * U, TAIL_VEC):
                    insert(accs[r], bufs[pb][rib, pl.ds(v * L, L)])

    acc_v = jnp.zeros((L,), jnp.float32)
    for r in range(4):
        rib = h * 4 + r
        lane = ((rg & 1) * 8 + rib).astype(jnp.int32)
        y_val = ybuf[pl.ds(lane, L)][0]
        pltpu.sync_copy(
            x_hbm.at[pl.ds(row_base, 8),
                     pl.ds(pl.multiple_of((y_val >> 7) << 7, 128), 128)],
            tbuf)
        ylo = y_val & 127
        st16 = (ylo >> 4) << 4
        xbuf[pl.ds(0, L)] = tbuf[rib, pl.ds(st16, L)]
        s_v = jnp.broadcast_to(xbuf[pl.ds(ylo & 15, L)][0], (L,))

        a = accs[r]
        csum_v = jnp.zeros((L,), jnp.float32)
        c_v = csum_v
        m_v = csum_v
        for j in range(6):
            m_v = bmax(a[0])
            c_v = jnp.maximum(m_v + 1.0 - s_v, 0.0)
            csum_v = csum_v + c_v
            if j < 5:
                eq = a[0] == m_v
                pm = lane_ids == bmin(jnp.where(eq, lane_ids, L))
                for k in range(5):
                    a[k] = jnp.where(pm, a[k + 1], a[k])
                a[5] = jnp.where(pm, jnp.full((L,), NEG, jnp.float32), a[5])
        sub_v = jnp.where(s_v >= m_v, jnp.full((L,), jnp.float32(1.0)), c_v)
        acc_v = acc_v + (csum_v - sub_v)

    obuf[...] = acc_v * jnp.float32(1.0 / (K * B))
    pltpu.sync_copy(obuf, out_hbm.at[pl.ds(pl.multiple_of(wid * L, 8), L)])


def kernel(x, y):
    parts = _loss_sc(x, y.astype(jnp.int32))
    return jnp.sum(parts.reshape(NW, L)[:, 0])


# speedup vs baseline: 1.7898x; 1.0626x over previous
import functools

import jax
import jax.numpy as jnp
from jax import lax
from jax.experimental import pallas as pl
from jax.experimental.pallas import tpu as pltpu
from jax.experimental.pallas import tpu_sc as plsc

B = 128
N = 100000
NPAD = 100096
K = 5
L = 16
NC = 2
NS = 16
NW = NC * NS
CH = 5888
NCHUNK = NPAD // CH
U = 8
FULL_IT = CH // (L * U)
TAIL_VEC = (N - (NCHUNK - 1) * CH) // L
NEG = -3.0e38

_mesh = plsc.VectorSubcoreMesh(core_axis_name="c", subcore_axis_name="s")


@functools.partial(
    pl.kernel,
    mesh=_mesh,
    out_type=jax.ShapeDtypeStruct((NW * L,), jnp.float32),
    scratch_types=[
        pltpu.VMEM((8, CH), jnp.float32),
        pltpu.VMEM((8, CH), jnp.float32),
        pltpu.VMEM((8, 128), jnp.float32),
        pltpu.VMEM((2 * L,), jnp.int32),
        pltpu.VMEM((2 * L,), jnp.float32),
        pltpu.VMEM((L,), jnp.float32),
        pltpu.SemaphoreType.DMA,
        pltpu.SemaphoreType.DMA,
    ],
)
def _loss_sc(x_hbm, y_hbm, out_hbm, buf0, buf1, tbuf, ybuf, xbuf, obuf,
             sem0, sem1):
    bufs = (buf0, buf1)
    sems = (sem0, sem1)
    cid = lax.axis_index("c")
    sid = lax.axis_index("s")
    wid = sid * NC + cid
    rg = cid * 8 + (sid >> 1)
    h = sid & 1
    lane_ids = lax.iota(jnp.int32, L)

    gdims = lax.GatherDimensionNumbers(
        offset_dims=(), collapsed_slice_dims=(0,), start_index_map=(0,))

    def shuf(v, s):
        return lax.gather(
            v, (lane_ids ^ s)[:, None], gdims, (1,),
            mode=lax.GatherScatterMode.PROMISE_IN_BOUNDS)

    def bmax(v):
        for s in (1, 2, 4, 8):
            v = jnp.maximum(v, shuf(v, s))
        return v

    def bmin(v):
        for s in (1, 2, 4, 8):
            v = jnp.minimum(v, shuf(v, s))
        return v

    ybase = pl.multiple_of((rg >> 1) * L, 16)
    pltpu.sync_copy(y_hbm.at[pl.ds(ybase, L)], ybuf.at[pl.ds(0, L)])

    def insert(a, t):
        for k in range(6):
            hi = jnp.maximum(a[k], t)
            t = jnp.minimum(a[k], t)
            a[k] = hi

    NET = ((0, 1), (2, 3), (4, 5), (6, 7), (0, 2), (1, 3), (4, 6), (5, 7),
           (1, 2), (5, 6), (0, 4), (1, 5), (2, 6), (3, 7), (2, 4), (3, 5),
           (1, 2), (3, 4), (5, 6))

    def chunk_body(pb, rib, i, carry):
        a = list(carry)
        base = i * (L * U)
        v = [bufs[pb][rib, pl.ds(base + u * L, L)] for u in range(U)]
        for i1, i2 in NET:
            hi = jnp.maximum(v[i1], v[i2])
            v[i2] = jnp.minimum(v[i1], v[i2])
            v[i1] = hi
        for j in range(6):
            t = v[j]
            for k in range(j, 6):
                hi = jnp.maximum(a[k], t)
                t = jnp.minimum(a[k], t)
                a[k] = hi
        return tuple(a)

    row_base = pl.multiple_of(rg * 8, 8)
    accs = [[jnp.full((L,), NEG, jnp.float32)] * 6 for _ in range(4)]

    pend = [None, None]
    pend[0] = pltpu.async_copy(
        x_hbm.at[pl.ds(row_base, 8), pl.ds(0, CH)], buf0, sems[0])
    for ci in range(NCHUNK):
        pb = ci % 2
        if ci + 1 < NCHUNK:
            nb = (ci + 1) % 2
            pend[nb] = pltpu.async_copy(
                x_hbm.at[pl.ds(row_base, 8),
                         pl.ds(pl.multiple_of((ci + 1) * CH, 128), CH)],
                bufs[nb], sems[nb])
        pend[pb].wait()
        n_it = FULL_IT if ci + 1 < NCHUNK else TAIL_VEC // U
        for r in range(4):
            rib = h * 4 + r
            accs[r] = list(lax.fori_loop(
                0, n_it, functools.partial(chunk_body, pb, rib),
                tuple(accs[r])))
            if ci + 1 == NCHUNK:
                for v in range(n_it * U, TAIL_VEC):
                    insert(accs[r], bufs[pb][rib, pl.ds(v * L, L)])

    acc_v = jnp.zeros((L,), jnp.float32)
    for r in range(4):
        rib = h * 4 + r
        lane = ((rg & 1) * 8 + rib).astype(jnp.int32)
        y_val = ybuf[pl.ds(lane, L)][0]
        pltpu.sync_copy(
            x_hbm.at[pl.ds(row_base, 8),
                     pl.ds(pl.multiple_of((y_val >> 7) << 7, 128), 128)],
            tbuf)
        ylo = y_val & 127
        st16 = (ylo >> 4) << 4
        xbuf[pl.ds(0, L)] = tbuf[rib, pl.ds(st16, L)]
        s_v = jnp.broadcast_to(xbuf[pl.ds(ylo & 15, L)][0], (L,))

        a = accs[r]
        csum_v = jnp.zeros((L,), jnp.float32)
        c_v = csum_v
        m_v = csum_v
        for j in range(6):
            m_v = bmax(a[0])
            c_v = jnp.maximum(m_v + 1.0 - s_v, 0.0)
            csum_v = csum_v + c_v
            if j < 5:
                eq = a[0] == m_v
                pm = lane_ids == bmin(jnp.where(eq, lane_ids, L))
                for k in range(5):
                    a[k] = jnp.where(pm, a[k + 1], a[k])
                a[5] = jnp.where(pm, jnp.full((L,), NEG, jnp.float32), a[5])
        sub_v = jnp.where(s_v >= m_v, jnp.full((L,), jnp.float32(1.0)), c_v)
        acc_v = acc_v + (csum_v - sub_v)

    obuf[...] = acc_v * jnp.float32(1.0 / (K * B))
    pltpu.sync_copy(obuf, out_hbm.at[pl.ds(pl.multiple_of(wid * L, 8), L)])


def kernel(x, y):
    parts = _loss_sc(x, y.astype(jnp.int32))
    return jnp.sum(parts.reshape(NW, L)[:, 0])


# speedup vs baseline: 1.8638x; 1.0414x over previous
import functools

import jax
import jax.numpy as jnp
from jax import lax
from jax.experimental import pallas as pl
from jax.experimental.pallas import tpu as pltpu
from jax.experimental.pallas import tpu_sc as plsc

B = 128
N = 100000
NPAD = 100096
K = 5
L = 16
NC = 2
NS = 16
NW = NC * NS
CH = 5888
NCHUNK = NPAD // CH
U = 16
FULL_IT = CH // (L * U)
TAIL_VEC = (N - (NCHUNK - 1) * CH) // L
NEG = -3.0e38

_mesh = plsc.VectorSubcoreMesh(core_axis_name="c", subcore_axis_name="s")


@functools.partial(
    pl.kernel,
    mesh=_mesh,
    out_type=jax.ShapeDtypeStruct((NW * L,), jnp.float32),
    scratch_types=[
        pltpu.VMEM((8, CH), jnp.float32),
        pltpu.VMEM((8, CH), jnp.float32),
        pltpu.VMEM((8, 128), jnp.float32),
        pltpu.VMEM((2 * L,), jnp.int32),
        pltpu.VMEM((2 * L,), jnp.float32),
        pltpu.VMEM((L,), jnp.float32),
        pltpu.SemaphoreType.DMA,
        pltpu.SemaphoreType.DMA,
    ],
)
def _loss_sc(x_hbm, y_hbm, out_hbm, buf0, buf1, tbuf, ybuf, xbuf, obuf,
             sem0, sem1):
    bufs = (buf0, buf1)
    sems = (sem0, sem1)
    cid = lax.axis_index("c")
    sid = lax.axis_index("s")
    wid = sid * NC + cid
    rg = cid * 8 + (sid >> 1)
    h = sid & 1
    lane_ids = lax.iota(jnp.int32, L)

    gdims = lax.GatherDimensionNumbers(
        offset_dims=(), collapsed_slice_dims=(0,), start_index_map=(0,))

    def shuf(v, s):
        return lax.gather(
            v, (lane_ids ^ s)[:, None], gdims, (1,),
            mode=lax.GatherScatterMode.PROMISE_IN_BOUNDS)

    def bmax(v):
        for s in (1, 2, 4, 8):
            v = jnp.maximum(v, shuf(v, s))
        return v

    def bmin(v):
        for s in (1, 2, 4, 8):
            v = jnp.minimum(v, shuf(v, s))
        return v

    ybase = pl.multiple_of((rg >> 1) * L, 16)
    pltpu.sync_copy(y_hbm.at[pl.ds(ybase, L)], ybuf.at[pl.ds(0, L)])

    def insert(a, t):
        for k in range(6):
            hi = jnp.maximum(a[k], t)
            t = jnp.minimum(a[k], t)
            a[k] = hi

    NET = ((0, 1), (2, 3), (4, 5), (6, 7), (0, 2), (1, 3), (4, 6), (5, 7),
           (1, 2), (5, 6), (0, 4), (1, 5), (2, 6), (3, 7), (2, 4), (3, 5),
           (1, 2), (3, 4), (5, 6))

    def block8(a, v):
        for i1, i2 in NET:
            hi = jnp.maximum(v[i1], v[i2])
            v[i2] = jnp.minimum(v[i1], v[i2])
            v[i1] = hi
        for j in range(6):
            t = v[j]
            for k in range(j, 6):
                hi = jnp.maximum(a[k], t)
                t = jnp.minimum(a[k], t)
                a[k] = hi

    def chunk_body(pb, rib, i, carry):
        a = list(carry[:6])
        b = list(carry[6:])
        base = i * (L * U)
        v = [bufs[pb][rib, pl.ds(base + u * L, L)] for u in range(U)]
        block8(a, v[:8])
        block8(b, v[8:])
        return tuple(a) + tuple(b)

    row_base = pl.multiple_of(rg * 8, 8)

    def dma_chunk(c, buf, sem):
        return pltpu.async_copy(
            x_hbm.at[pl.ds(row_base, 8),
                     pl.ds(pl.multiple_of(c * CH, 128), CH)],
            buf, sem)

    def wait_chunk(c, buf, sem):
        pltpu.make_async_copy(
            x_hbm.at[pl.ds(row_base, 8),
                     pl.ds(pl.multiple_of(c * CH, 128), CH)],
            buf, sem).wait()

    def process(pb, sts, n_it):
        sts = list(sts)
        for r in range(4):
            sts[r] = lax.fori_loop(
                0, n_it, functools.partial(chunk_body, pb, h * 4 + r),
                tuple(sts[r]))
        return sts

    accs = [tuple([jnp.full((L,), NEG, jnp.float32)] * 12) for _ in range(4)]

    dma_chunk(0, buf0, sem0)

    def ring_body(i, sts):
        wait_chunk(2 * i, buf0, sem0)
        dma_chunk(2 * i + 1, buf1, sem1)
        sts = process(0, sts, FULL_IT)
        wait_chunk(2 * i + 1, buf1, sem1)
        dma_chunk(2 * i + 2, buf0, sem0)
        sts = process(1, sts, FULL_IT)
        return tuple(sts)

    accs = list(lax.fori_loop(0, (NCHUNK - 1) // 2, ring_body, tuple(accs)))

    wait_chunk(NCHUNK - 1, buf0, sem0)
    accs = process(0, accs, TAIL_VEC // U)
    for r in range(4):
        a = list(accs[r])
        rib = h * 4 + r
        for v in range((TAIL_VEC // U) * U, TAIL_VEC):
            insert(a, buf0[rib, pl.ds(v * L, L)])
        accs[r] = tuple(a)

    acc_v = jnp.zeros((L,), jnp.float32)
    for r in range(4):
        rib = h * 4 + r
        lane = ((rg & 1) * 8 + rib).astype(jnp.int32)
        y_val = ybuf[pl.ds(lane, L)][0]
        pltpu.sync_copy(
            x_hbm.at[pl.ds(row_base, 8),
                     pl.ds(pl.multiple_of((y_val >> 7) << 7, 128), 128)],
            tbuf)
        ylo = y_val & 127
        st16 = (ylo >> 4) << 4
        xbuf[pl.ds(0, L)] = tbuf[rib, pl.ds(st16, L)]
        s_v = jnp.broadcast_to(xbuf[pl.ds(ylo & 15, L)][0], (L,))

        a = list(accs[r][:6])
        bstk = list(accs[r][6:])
        for j in range(6):
            t = bstk[j]
            for k in range(j, 6):
                hi = jnp.maximum(a[k], t)
                t = jnp.minimum(a[k], t)
                a[k] = hi
        csum_v = jnp.zeros((L,), jnp.float32)
        c_v = csum_v
        m_v = csum_v
        for j in range(6):
            m_v = bmax(a[0])
            c_v = jnp.maximum(m_v + 1.0 - s_v, 0.0)
            csum_v = csum_v + c_v
            if j < 5:
                eq = a[0] == m_v
                pm = lane_ids == bmin(jnp.where(eq, lane_ids, L))
                for k in range(5):
                    a[k] = jnp.where(pm, a[k + 1], a[k])
                a[5] = jnp.where(pm, jnp.full((L,), NEG, jnp.float32), a[5])
        sub_v = jnp.where(s_v >= m_v, jnp.full((L,), jnp.float32(1.0)), c_v)
        acc_v = acc_v + (csum_v - sub_v)

    obuf[...] = acc_v * jnp.float32(1.0 / (K * B))
    pltpu.sync_copy(obuf, out_hbm.at[pl.ds(pl.multiple_of(wid * L, 8), L)])


def kernel(x, y):
    parts = _loss_sc(x, y.astype(jnp.int32))
    return jnp.sum(parts.reshape(NW, L)[:, 0])
